# baseline (speedup 1.0000x reference)
import functools

import jax
import jax.numpy as jnp
from jax import lax
from jax.experimental import pallas as pl
from jax.experimental.pallas import tpu as pltpu
from jax.experimental.pallas import tpu_sc as plsc

LATENT = 32
BATCH = 16384
NC = 2
NS = 16
NW = NC * NS
B_PER_W = BATCH // NW
IDS_ROWS = B_PER_W // 128


def _nmf_body(uid_hbm, iid_hbm, uf_hbm, if_hbm, out_hbm,
              uid_v, iid_v, urows, irows, out_v, sem_u, sem_i):
    wid = lax.axis_index("s") * NC + lax.axis_index("c")
    row0 = wid * IDS_ROWS
    base = wid * B_PER_W

    pltpu.sync_copy(uid_hbm.at[pl.ds(row0, IDS_ROWS)], uid_v)
    pltpu.sync_copy(iid_hbm.at[pl.ds(row0, IDS_ROWS)], iid_v)

    copies = []
    for j in range(IDS_ROWS):
        copies.append(pltpu.async_copy(
            uf_hbm.at[uid_v.at[j]], urows.at[pl.ds(j * 128, 128)], sem_u))
        copies.append(pltpu.async_copy(
            if_hbm.at[iid_v.at[j]], irows.at[pl.ds(j * 128, 128)], sem_i))
    for c in copies:
        c.wait()

    lane = lax.iota(jnp.int32, 16)

    def group(g, _):
        rows = g * 16 + lane
        acc = jnp.zeros((16,), jnp.float32)
        for d in range(LATENT):
            col = jnp.full((16,), d, jnp.int32)
            uc = plsc.load_gather(urows, [rows, col])
            ic = plsc.load_gather(irows, [rows, col])
            acc = acc + uc * ic
        out_v[pl.ds(g * 16, 16)] = acc
        return 0

    lax.fori_loop(0, B_PER_W // 16, group, 0)

    pltpu.sync_copy(out_v, out_hbm.at[pl.ds(base, B_PER_W)])


def kernel(user_ids, item_ids, user_factors, item_factors):
    uid2d = user_ids.reshape(BATCH // 128, 128)
    iid2d = item_ids.reshape(BATCH // 128, 128)
    mesh = plsc.VectorSubcoreMesh(core_axis_name="c", subcore_axis_name="s")
    run = functools.partial(
        pl.kernel, mesh=mesh,
        out_type=jax.ShapeDtypeStruct((BATCH,), jnp.float32),
        compiler_params=pltpu.CompilerParams(
            use_tc_tiling_on_sc=False, needs_layout_passes=False),
        scratch_types=[
            pltpu.VMEM((IDS_ROWS, 128), jnp.int32),
            pltpu.VMEM((IDS_ROWS, 128), jnp.int32),
            pltpu.VMEM((B_PER_W, LATENT), jnp.float32),
            pltpu.VMEM((B_PER_W, LATENT), jnp.float32),
            pltpu.VMEM((B_PER_W,), jnp.float32),
            pltpu.SemaphoreType.DMA,
            pltpu.SemaphoreType.DMA,
        ],
    )(_nmf_body)
    return run(uid2d, iid2d, user_factors, item_factors)


# speedup vs baseline: 1.4932x; 1.4932x over previous
import functools

import jax
import jax.numpy as jnp
from jax import lax
from jax.experimental import pallas as pl
from jax.experimental.pallas import tpu as pltpu
from jax.experimental.pallas import tpu_sc as plsc

LATENT = 32
BATCH = 16384
NC = 2
NS = 16
NW = NC * NS
B_PER_W = BATCH // NW
CHUNK = 128
NCHUNK = B_PER_W // CHUNK


def _nmf_body(uid_hbm, iid_hbm, uf_hbm, if_hbm, out_hbm,
              uid_v, iid_v, ubuf0, ubuf1, ibuf0, ibuf1, out_v,
              sem_u, sem_i, sem_out):
    wid = lax.axis_index("s") * NC + lax.axis_index("c")
    base = wid * B_PER_W

    pltpu.sync_copy(uid_hbm.at[pl.ds(base, B_PER_W)], uid_v)
    pltpu.sync_copy(iid_hbm.at[pl.ds(base, B_PER_W)], iid_v)

    lane = lax.iota(jnp.int32, 16)
    ubufs = (ubuf0, ubuf1)
    ibufs = (ibuf0, ibuf1)

    def issue(c, slot):
        ub = ubufs[slot]
        ib = ibufs[slot]

        def grp(g, _):
            iu = uid_v[pl.ds(c * CHUNK + g * 16, 16)]
            ii = iid_v[pl.ds(c * CHUNK + g * 16, 16)]
            for k in range(16):
                b = g * 16 + k
                pltpu.async_copy(uf_hbm.at[pl.ds(iu[k], 1)],
                                 ub.at[pl.ds(b, 1)], sem_u)
                pltpu.async_copy(if_hbm.at[pl.ds(ii[k], 1)],
                                 ib.at[pl.ds(b, 1)], sem_i)
            return 0

        lax.fori_loop(0, CHUNK // 16, grp, 0)

    def drain(sem, buf):
        pltpu.make_async_copy(uf_hbm.at[pl.ds(0, CHUNK)], buf, sem).wait()

    def compute(c, slot):
        ub = ubufs[slot]
        ib = ibufs[slot]

        def group(g, _):
            rows = g * 16 + lane
            acc = jnp.zeros((16,), jnp.float32)
            for d in range(LATENT):
                col = jnp.full((16,), d, jnp.int32)
                uc = plsc.load_gather(ub, [rows, col])
                ic = plsc.load_gather(ib, [rows, col])
                acc = acc + uc * ic
            out_v[pl.ds(c * CHUNK + g * 16, 16)] = acc
            return 0

        lax.fori_loop(0, CHUNK // 16, group, 0)

    issue(0, 0)
    for c in range(NCHUNK):
        if c + 1 < NCHUNK:
            issue(c + 1, (c + 1) % 2)
        drain(sem_u, ubufs[c % 2])
        drain(sem_i, ibufs[c % 2])
        compute(c, c % 2)

    pltpu.async_copy(out_v, out_hbm.at[pl.ds(base, B_PER_W)], sem_out).wait()


def kernel(user_ids, item_ids, user_factors, item_factors):
    mesh = plsc.VectorSubcoreMesh(core_axis_name="c", subcore_axis_name="s")
    run = functools.partial(
        pl.kernel, mesh=mesh,
        out_type=jax.ShapeDtypeStruct((BATCH,), jnp.float32),
        compiler_params=pltpu.CompilerParams(needs_layout_passes=False),
        scratch_types=[
            pltpu.VMEM((B_PER_W,), jnp.int32),
            pltpu.VMEM((B_PER_W,), jnp.int32),
            pltpu.VMEM((CHUNK, LATENT), jnp.float32),
            pltpu.VMEM((CHUNK, LATENT), jnp.float32),
            pltpu.VMEM((CHUNK, LATENT), jnp.float32),
            pltpu.VMEM((CHUNK, LATENT), jnp.float32),
            pltpu.VMEM((B_PER_W,), jnp.float32),
            pltpu.SemaphoreType.DMA,
            pltpu.SemaphoreType.DMA,
            pltpu.SemaphoreType.DMA,
        ],
    )(_nmf_body)
    return run(user_ids, item_ids, user_factors, item_factors)
